# baseline (speedup 1.0000x reference)
import functools

import jax
import jax.numpy as jnp
from jax import lax
from jax.experimental import pallas as pl
from jax.experimental.pallas import tpu as pltpu
from jax.experimental.pallas import tpu_sc as plsc

MAX_REL = 100
EMB = 64
IDS_MINOR = 128


@functools.lru_cache(maxsize=None)
def _build(n_ids_rows: int, n_table_rows: int):
    info = plsc.get_sparse_core_info()
    L = info.num_lanes
    num_workers = info.num_cores * info.num_subcores
    rows_per_worker = n_ids_rows // num_workers
    n_blocks = IDS_MINOR // L
    table_elems = n_table_rows * EMB
    chunk_elems = IDS_MINOR * EMB

    mesh = plsc.VectorSubcoreMesh(core_axis_name="c", subcore_axis_name="s")

    @functools.partial(
        pl.kernel,
        mesh=mesh,
        out_type=jax.ShapeDtypeStruct((n_ids_rows * IDS_MINOR * EMB,), jnp.float32),
        scratch_types=[
            pltpu.VMEM((rows_per_worker, IDS_MINOR), jnp.int32),
            pltpu.VMEM((table_elems,), jnp.float32),
            pltpu.VMEM((chunk_elems,), jnp.float32),
            pltpu.VMEM((chunk_elems,), jnp.float32),
            pltpu.SemaphoreType.DMA,
            pltpu.SemaphoreType.DMA,
        ],
        compiler_params=pltpu.CompilerParams(
            use_tc_tiling_on_sc=False, needs_layout_passes=False
        ),
    )
    def k(ids_hbm, w_hbm, out_hbm, idx_v, table_v, rows0, rows1, osem0, osem1):
        wid = lax.axis_index("s") * info.num_cores + lax.axis_index("c")
        row0 = wid * rows_per_worker
        out0 = row0 * chunk_elems
        rows_bufs = (rows0, rows1)
        osems = (osem0, osem1)

        pltpu.sync_copy(w_hbm, table_v)
        pltpu.sync_copy(ids_hbm.at[pl.ds(row0, rows_per_worker)], idx_v)

        def clamp_row(r, carry):
            for kk in range(IDS_MINOR // L):
                sl = pl.ds(kk * L, L)
                idx_v[r, sl] = jnp.minimum(idx_v[r, sl], MAX_REL)
            return carry

        lax.fori_loop(0, rows_per_worker, clamp_row, 0)

        lanes = jax.lax.iota(jnp.int32, L)
        colv = [((lanes + c) & (L - 1)) for c in range(L)]
        dstv = [lanes * EMB + colv[c] for c in range(L)]

        def assemble_chunk(ch, buf):
            ivecs = [idx_v[ch, pl.ds(b * L, L)] for b in range(n_blocks)]
            rowshift = [iv * EMB for iv in ivecs]
            for g4 in range(EMB // L):
                src_ref = table_v.at[pl.ds(g4 * L, table_elems - EMB + L)]
                for c in range(L):
                    gs = [
                        plsc.load_gather(src_ref, [rowshift[b] + colv[c]])
                        for b in range(n_blocks)
                    ]
                    for b in range(n_blocks):
                        plsc.store_scatter(
                            buf.at[
                                pl.ds(b * L * EMB + g4 * L, (L - 1) * EMB + L)
                            ],
                            [dstv[c]],
                            gs[b],
                        )

        def writeback(ch, buf, sem):
            return pltpu.make_async_copy(
                buf,
                out_hbm.at[pl.ds(out0 + ch * chunk_elems, chunk_elems)],
                sem,
            )

        for b in (0, 1):
            assemble_chunk(b, rows_bufs[b])
            writeback(b, rows_bufs[b], osems[b]).start()

        def body(g, carry):
            for b in (0, 1):
                ch = 2 * g + b
                writeback(ch - 2, rows_bufs[b], osems[b]).wait()
                assemble_chunk(ch, rows_bufs[b])
                writeback(ch, rows_bufs[b], osems[b]).start()
            return carry

        lax.fori_loop(1, rows_per_worker // 2, body, 0)

        for b in (0, 1):
            writeback(rows_per_worker - 2 + b, rows_bufs[b], osems[b]).wait()

    return k


def kernel(relative_position_ids, weight):
    b, h = relative_position_ids.shape
    ids2 = relative_position_ids.astype(jnp.int32).reshape(-1, IDS_MINOR)
    out = _build(ids2.shape[0], weight.shape[0])(ids2, weight.reshape(-1))
    return out.reshape(b, h, EMB)
